# baseline (speedup 1.0000x reference)
import dataclasses
import functools

import jax
import jax.numpy as jnp
from jax import lax
from jax.experimental import pallas as pl
from jax.experimental.pallas import tpu as pltpu
from jax.experimental.pallas import tpu_sc as plsc

B, N, D, E = 4, 10000, 128, 320000
HEADS = 4
HID = 128
C = HID // HEADS
NT = B * N

ROW_BLK = 1000

NCORE = 2
NSUB = 16
NWORK = NCORE * NSUB
K = 384
ETOT = B * E + NT
EP = ((ETOT + NWORK * K - 1) // (NWORK * K)) * (NWORK * K)
R = 40960
SPARE = NT
STRIPE = R // NSUB


def _dense_stage_kernel(h_ref, w_ref, a_ref, xl_ref, al_ref):
    xl = jnp.dot(h_ref[...], w_ref[...], preferred_element_type=jnp.float32)
    xl_ref[...] = xl
    al_ref[...] = jnp.dot(xl, a_ref[...], preferred_element_type=jnp.float32)


def _dense_stage(h, w, a_pack):
    nrows = h.shape[0]
    grid = (nrows // ROW_BLK,)
    return pl.pallas_call(
        _dense_stage_kernel,
        grid=grid,
        in_specs=[
            pl.BlockSpec((ROW_BLK, D), lambda i: (i, 0)),
            pl.BlockSpec((D, HID), lambda i: (0, 0)),
            pl.BlockSpec((HID, 128), lambda i: (0, 0)),
        ],
        out_specs=[
            pl.BlockSpec((ROW_BLK, HID), lambda i: (i, 0)),
            pl.BlockSpec((ROW_BLK, 128), lambda i: (i, 0)),
        ],
        out_shape=[
            jax.ShapeDtypeStruct((nrows, HID), jnp.float32),
            jax.ShapeDtypeStruct((nrows, 128), jnp.float32),
        ],
    )(h, w, a_pack)


def _pack_attn(a_s, a_d):
    eye = jnp.eye(HEADS, dtype=jnp.float32)
    blk_s = (a_s.reshape(HEADS, C)[:, :, None] * eye[:, None, :]).reshape(HID, HEADS)
    blk_d = (a_d.reshape(HEADS, C)[:, :, None] * eye[:, None, :]).reshape(HID, HEADS)
    return jnp.concatenate(
        [blk_s, blk_d, jnp.zeros((HID, 128 - 2 * HEADS), jnp.float32)], axis=1
    )


_MESH = plsc.VectorSubcoreMesh(
    core_axis_name="c", subcore_axis_name="s", num_cores=NCORE, num_subcores=NSUB
)

_NBLK_A = EP // (NWORK * K)

_SC_PARAMS = pltpu.CompilerParams(
    needs_layout_passes=False, use_tc_tiling_on_sc=False
)


def _edge_ex_kernel(s_hbm, d_hbm, as_hbm, ad_hbm, bnd_hbm, z16_hbm,
                    ex_hbm, den_hbm,
                    sidx, didx, asr, adr, exc, exr, bnd,
                    semL, semG, semE, den_acc):
    cid = lax.axis_index("c")
    sid = lax.axis_index("s")
    wid = sid * NCORE + cid
    nb = _NBLK_A
    bufs = [(sidx[b], didx[b], asr[b], adr[b], exc[b], semL[b], semG[b],
             semE[b]) for b in range(NBUF)]

    pltpu.sync_copy(z16_hbm, den_acc.at[pl.ds(sid * STRIPE, STRIPE)])
    pltpu.sync_copy(bnd_hbm, bnd)
    plsc.subcore_barrier()

    lanes = lax.iota(jnp.int32, 16)
    row_pat = lanes >> 2
    col_pat = lanes & 3
    bv = bnd[...]

    def issue_loads(t, buf):
        (si, di, sem) = (buf[0], buf[1], buf[5])
        bs = wid * (nb * K) + t * K
        pltpu.async_copy(s_hbm.at[pl.ds(bs, K)], si, sem)
        pltpu.async_copy(d_hbm.at[pl.ds(bs, K)], di.at[0], sem)

    def wait_loads(buf):
        (si, di, sem) = (buf[0], buf[1], buf[5])
        pltpu.make_async_copy(s_hbm.at[pl.ds(0, K)], si, sem).wait()
        pltpu.make_async_copy(d_hbm.at[pl.ds(0, K)], di.at[0], sem).wait()

    def issue_gathers(buf):
        (si, di, ar, dr, sem) = (buf[0], buf[1], buf[2], buf[3], buf[6])
        pltpu.async_copy(as_hbm.at[si], ar, sem)
        pltpu.async_copy(ad_hbm.at[di.at[0]], dr, sem)

    def wait_gathers(buf):
        (si, di, ar, dr, sem) = (buf[0], buf[1], buf[2], buf[3], buf[6])
        pltpu.make_async_copy(as_hbm.at[si], ar, sem).wait()
        pltpu.make_async_copy(ad_hbm.at[di.at[0]], dr, sem).wait()

    def body(t, b):
        cur = bufs[b]
        (si, di, ar, dr, ec, semL_, semG_, semE_) = cur
        wait_gathers(cur)

        @pl.loop(0, K, unroll=16)
        def _(j):
            a = ar[j] + dr[j]
            t_ = jnp.maximum(a, a * 0.2)
            exr[j] = jnp.exp(t_ - bv)

        pltpu.sync_copy(exr, den_acc.at[di.at[0]], add=True)

        @pl.when(t >= NBUF)
        def _():
            pltpu.make_async_copy(ec, ex_hbm.at[pl.ds(0, 4 * K)], semE_).wait()

        @pl.loop(0, K // 4, unroll=4)
        def _(g):
            v = plsc.load_gather(exr, [4 * g + row_pat, col_pat])
            ec[pl.ds(16 * g, 16)] = v

        bs = wid * (nb * K) + t * K
        pltpu.async_copy(ec, ex_hbm.at[pl.ds(4 * bs, 4 * K)], semE_)

        @pl.when(t + NBUF < nb)
        def _():
            issue_loads(t + NBUF, cur)

        @pl.when(t + NBUF - 1 < nb)
        def _():
            nxt = bufs[(b + NBUF - 1) % NBUF]
            wait_loads(nxt)
            issue_gathers(nxt)

    for b in range(NBUF):
        issue_loads(b, bufs[b])
    for b in range(NBUF - 1):
        wait_loads(bufs[b])
        issue_gathers(bufs[b])

    @pl.loop(0, nb // NBUF)
    def _(tp):
        for b in range(NBUF):
            body(NBUF * tp + b, b)

    for b in range(NBUF):
        pltpu.make_async_copy(exc[b], ex_hbm.at[pl.ds(0, 4 * K)],
                              semE[b]).wait()

    plsc.subcore_barrier()
    off = sid * STRIPE
    pltpu.sync_copy(den_acc.at[pl.ds(off, STRIPE)],
                    den_hbm.at[pl.ds(cid * R + off, STRIPE)])


@jax.jit
def _edge_ex(s, d, as_tab, ad_tab, bound, z16):
    return pl.kernel(
        _edge_ex_kernel,
        out_type=[
            jax.ShapeDtypeStruct((4 * EP,), jnp.float32),
            jax.ShapeDtypeStruct((NCORE * R, 16), jnp.float32),
        ],
        mesh=_MESH,
        scratch_types=[
            [pltpu.VMEM((K,), jnp.int32) for _ in range(NBUF)],
            [pltpu.VMEM((1, K), jnp.int32) for _ in range(NBUF)],
            [pltpu.VMEM((K, 16), jnp.float32) for _ in range(NBUF)],
            [pltpu.VMEM((K, 16), jnp.float32) for _ in range(NBUF)],
            [pltpu.VMEM((4 * K,), jnp.float32) for _ in range(NBUF)],
            pltpu.VMEM((K, 16), jnp.float32),
            pltpu.VMEM((16,), jnp.float32),
            [pltpu.SemaphoreType.DMA for _ in range(NBUF)],
            [pltpu.SemaphoreType.DMA for _ in range(NBUF)],
            [pltpu.SemaphoreType.DMA for _ in range(NBUF)],
            pltpu.VMEM_SHARED((R, 16), jnp.float32),
        ],
        compiler_params=_SC_PARAMS,
    )(s, d, as_tab, ad_tab, bound, z16)


KB = 256
NBUF = 4
_NBLK_B = EP // (NSUB * KB)


def _aggregate_kernel(s_hbm, d_hbm, ex_hbm, x0_hbm, x1_hbm, x2_hbm, x3_hbm,
                      z32_hbm, out_hbm, sidx, didx, msg, exb, semL, semG,
                      out_acc):
    cid = lax.axis_index("c")
    sid = lax.axis_index("s")
    nb = _NBLK_B
    off = sid * STRIPE
    bufs = [(sidx[b], didx[b], msg[b], exb[b], semL[b], semG[b])
            for b in range(NBUF)]

    def issue_loads(t, buf):
        (si, di, eb, sem) = (buf[0], buf[1], buf[3], buf[4])
        bs = sid * (nb * KB) + t * KB
        pltpu.async_copy(s_hbm.at[pl.ds(bs, KB)], si, sem)
        pltpu.async_copy(d_hbm.at[pl.ds(bs, KB)], di.at[0], sem)
        pltpu.async_copy(ex_hbm.at[pl.ds(4 * bs, 4 * KB)], eb, sem)

    def wait_loads(buf):
        (si, di, eb, sem) = (buf[0], buf[1], buf[3], buf[4])
        pltpu.make_async_copy(s_hbm.at[pl.ds(0, KB)], si, sem).wait()
        pltpu.make_async_copy(d_hbm.at[pl.ds(0, KB)], di.at[0], sem).wait()
        pltpu.make_async_copy(ex_hbm.at[pl.ds(0, 4 * KB)], eb, sem).wait()

    def one_pass(hbase, xa_hbm, xb_hbm):
        head = hbase + cid

        def issue_gather(buf):
            (si, mg, sem) = (buf[0], buf[2], buf[5])

            @pl.when(cid == 0)
            def _():
                pltpu.async_copy(xa_hbm.at[si], mg, sem)

            @pl.when(cid == 1)
            def _():
                pltpu.async_copy(xb_hbm.at[si], mg, sem)

        def wait_gather(buf):
            (si, mg, sem) = (buf[0], buf[2], buf[5])
            pltpu.make_async_copy(xa_hbm.at[si], mg, sem).wait()

        def body(t, b):
            cur = bufs[b]
            (si, di, mg, eb, semL_, semG_) = cur
            wait_gather(cur)

            @pl.loop(0, KB, unroll=16)
            def _(j):
                ebc = plsc.load_gather(
                    eb, [jnp.full((16,), 4 * j, jnp.int32) + head])
                mg[j, pl.ds(0, 16)] = mg[j, pl.ds(0, 16)] * ebc
                mg[j, pl.ds(16, 16)] = mg[j, pl.ds(16, 16)] * ebc

            pltpu.sync_copy(mg, out_acc.at[di.at[0]], add=True)

            @pl.when(t + NBUF < nb)
            def _():
                issue_loads(t + NBUF, cur)

            @pl.when(t + NBUF - 1 < nb)
            def _():
                nxt = bufs[(b + NBUF - 1) % NBUF]
                wait_loads(nxt)
                issue_gather(nxt)

        for b in range(NBUF):
            issue_loads(b, bufs[b])
        for b in range(NBUF - 1):
            wait_loads(bufs[b])
            issue_gather(bufs[b])

        @pl.loop(0, nb // NBUF)
        def _(tp):
            for b in range(NBUF):
                body(NBUF * tp + b, b)

        plsc.subcore_barrier()
        pltpu.sync_copy(out_acc.at[pl.ds(off, STRIPE)],
                        out_hbm.at[pl.ds((hbase + cid) * R + off, STRIPE)])
        plsc.subcore_barrier()

    pltpu.sync_copy(z32_hbm, out_acc.at[pl.ds(off, STRIPE)])
    plsc.subcore_barrier()
    one_pass(0, x0_hbm, x1_hbm)
    pltpu.sync_copy(z32_hbm, out_acc.at[pl.ds(off, STRIPE)])
    plsc.subcore_barrier()
    one_pass(2, x2_hbm, x3_hbm)


@jax.jit
def _aggregate(s, d, ex, x0, x1, x2, x3, z32):
    return pl.kernel(
        _aggregate_kernel,
        out_type=jax.ShapeDtypeStruct((HEADS * R, 32), jnp.float32),
        mesh=_MESH,
        scratch_types=[
            [pltpu.VMEM((KB,), jnp.int32) for _ in range(NBUF)],
            [pltpu.VMEM((1, KB), jnp.int32) for _ in range(NBUF)],
            [pltpu.VMEM((KB, 32), jnp.float32) for _ in range(NBUF)],
            [pltpu.VMEM((4 * KB,), jnp.float32) for _ in range(NBUF)],
            [pltpu.SemaphoreType.DMA for _ in range(NBUF)],
            [pltpu.SemaphoreType.DMA for _ in range(NBUF)],
            pltpu.VMEM_SHARED((R, 32), jnp.float32),
        ],
        compiler_params=_SC_PARAMS,
    )(s, d, ex, x0, x1, x2, x3, z32)


def _gat_layer(h, s, d, p, i, z16, z32):
    w, a_s, a_d, b = p[f"W{i}"], p[f"as{i}"], p[f"ad{i}"], p[f"b{i}"]
    xl, al = _dense_stage(h, w, _pack_attn(a_s, a_d))

    al_sd = al[:, : 2 * HEADS]
    colmax = al_sd.max(axis=0)
    z = colmax[:HEADS] + colmax[HEADS : 2 * HEADS]
    bound4 = jnp.maximum(z, 0.2 * z)
    bound = jnp.concatenate([bound4, jnp.full((12,), 88.0, jnp.float32)])

    pad_rows = jnp.zeros((R - NT, 16), jnp.float32)
    as_tab = jnp.concatenate(
        [al[:, :HEADS], jnp.zeros((NT, 16 - HEADS), jnp.float32)], axis=1)
    as_tab = jnp.concatenate([as_tab, pad_rows], axis=0)
    ad_tab = jnp.concatenate(
        [al[:, HEADS : 2 * HEADS], jnp.zeros((NT, 16 - HEADS), jnp.float32)], axis=1)
    ad_tab = jnp.concatenate([ad_tab, pad_rows], axis=0)

    ex, den_parts = _edge_ex(s, d, as_tab, ad_tab, bound, z16)
    den = den_parts[:R][:NT, :HEADS] + den_parts[R:][:NT, :HEADS]

    xlh = xl.reshape(NT, HEADS, C).transpose(1, 0, 2)
    xlh = jnp.concatenate(
        [xlh, jnp.zeros((HEADS, R - NT, C), jnp.float32)], axis=1)

    o = _aggregate(s, d, ex, xlh[0], xlh[1], xlh[2], xlh[3], z32)

    inv = 1.0 / (den + 1e-16)
    heads = [o[h * R : h * R + NT] * inv[:, h : h + 1] for h in range(HEADS)]
    return jnp.concatenate(heads, axis=1) + b


def _gnorm_relu(h, g, bt, ms, res=None):
    hb = h.reshape(B, N, -1)
    mean = hb.mean(axis=1, keepdims=True)
    hc = hb - ms * mean
    var = (hc * hc).mean(axis=1, keepdims=True)
    out = g * hc / jnp.sqrt(var + 1e-5) + bt
    out = out.reshape(NT, -1)
    if res is not None:
        out = out + res
    return jax.nn.relu(out)


def kernel(x, edge_index, params):
    p = params
    xt = jnp.transpose(x, (0, 2, 1)).reshape(NT, D)
    off = jnp.arange(B, dtype=edge_index.dtype) * N
    src = (edge_index[0][None, :] + off[:, None]).reshape(-1)
    dst = (edge_index[1][None, :] + off[:, None]).reshape(-1)
    loop = jnp.arange(NT, dtype=src.dtype)
    pad = jnp.full((EP - ETOT,), SPARE, jnp.int32)
    s = jnp.concatenate([src, loop, pad])
    d = jnp.concatenate([dst, loop, pad])
    z16 = jnp.zeros((STRIPE, 16), jnp.float32)
    z32 = jnp.zeros((STRIPE, 32), jnp.float32)

    h = _gat_layer(xt, s, d, p, 1, z16, z32)
    h = _gnorm_relu(h, p["g1"], p["bt1"], p["ms1"])
    h0 = h
    h = _gat_layer(h0, s, d, p, 2, z16, z32)
    h = _gnorm_relu(h, p["g2"], p["bt2"], p["ms2"], res=h0)
    h0 = h
    h = _gat_layer(h0, s, d, p, 3, z16, z32)
    h = _gnorm_relu(h, p["g3"], p["bt3"], p["ms3"], res=h0)

    pooled = h.reshape(B, N, HID).sum(axis=1) / float(N)
    return pooled @ p["Wc"] + p["bc"]


# speedup vs baseline: 1.0208x; 1.0208x over previous
import dataclasses
import functools

import jax
import jax.numpy as jnp
from jax import lax
from jax.experimental import pallas as pl
from jax.experimental.pallas import tpu as pltpu
from jax.experimental.pallas import tpu_sc as plsc

B, N, D, E = 4, 10000, 128, 320000
HEADS = 4
HID = 128
C = HID // HEADS
NT = B * N

ROW_BLK = 1024

NCORE = 2
NSUB = 16
NWORK = NCORE * NSUB
K = 384
ETOT = B * E + NT
EP = ((ETOT + NWORK * K - 1) // (NWORK * K)) * (NWORK * K)
R = 40960
SPARE = NT
STRIPE = R // NSUB


def _dense_stage_kernel(h_ref, w_ref, a_ref,
                        x0_ref, x1_ref, x2_ref, x3_ref, as_ref, ad_ref):
    xl = jnp.dot(h_ref[...], w_ref[...], preferred_element_type=jnp.float32)
    x0_ref[...] = xl[:, 0:32]
    x1_ref[...] = xl[:, 32:64]
    x2_ref[...] = xl[:, 64:96]
    x3_ref[...] = xl[:, 96:128]
    al = jnp.dot(xl, a_ref[...], preferred_element_type=jnp.float32)
    as_ref[...] = al[:, 0:16]
    ad_ref[...] = al[:, 16:32]


def _dense_stage(h, w, a_pack):
    grid = (R // ROW_BLK,)
    nrows = h.shape[0]
    hp = jnp.concatenate([h, jnp.zeros((R - nrows, D), jnp.float32)], axis=0)
    out32 = pl.BlockSpec((ROW_BLK, 32), lambda i: (i, 0))
    out16 = pl.BlockSpec((ROW_BLK, 16), lambda i: (i, 0))
    return pl.pallas_call(
        _dense_stage_kernel,
        grid=grid,
        in_specs=[
            pl.BlockSpec((ROW_BLK, D), lambda i: (i, 0)),
            pl.BlockSpec((D, HID), lambda i: (0, 0)),
            pl.BlockSpec((HID, 32), lambda i: (0, 0)),
        ],
        out_specs=[out32, out32, out32, out32, out16, out16],
        out_shape=[jax.ShapeDtypeStruct((R, 32), jnp.float32)] * 4
        + [jax.ShapeDtypeStruct((R, 16), jnp.float32)] * 2,
    )(hp, w, a_pack)


def _pack_attn(a_s, a_d):
    eye = jnp.eye(HEADS, dtype=jnp.float32)
    blk_s = (a_s.reshape(HEADS, C)[:, :, None] * eye[:, None, :]).reshape(HID, HEADS)
    blk_d = (a_d.reshape(HEADS, C)[:, :, None] * eye[:, None, :]).reshape(HID, HEADS)
    z12 = jnp.zeros((HID, 12), jnp.float32)
    return jnp.concatenate([blk_s, z12, blk_d, z12], axis=1)


_MESH = plsc.VectorSubcoreMesh(
    core_axis_name="c", subcore_axis_name="s", num_cores=NCORE, num_subcores=NSUB
)

_NBLK_A = EP // (NWORK * K)

_SC_PARAMS = pltpu.CompilerParams(
    needs_layout_passes=False, use_tc_tiling_on_sc=False
)


def _edge_ex_kernel(s_hbm, d_hbm, as_hbm, ad_hbm, bnd_hbm, z16_hbm,
                    ex_hbm, den_hbm,
                    sidx, didx, asr, adr, exc, exr, bnd,
                    semL, semG, semE, den_acc):
    cid = lax.axis_index("c")
    sid = lax.axis_index("s")
    wid = sid * NCORE + cid
    nb = _NBLK_A
    bufs = [(sidx[b], didx[b], asr[b], adr[b], exc[b], semL[b], semG[b],
             semE[b]) for b in range(NBUF)]

    pltpu.sync_copy(z16_hbm, den_acc.at[pl.ds(sid * STRIPE, STRIPE)])
    pltpu.sync_copy(bnd_hbm, bnd)
    plsc.subcore_barrier()

    lanes = lax.iota(jnp.int32, 16)
    row_pat = lanes >> 2
    col_pat = lanes & 3
    bv = bnd[...]

    def issue_loads(t, buf):
        (si, di, sem) = (buf[0], buf[1], buf[5])
        bs = wid * (nb * K) + t * K
        pltpu.async_copy(s_hbm.at[pl.ds(bs, K)], si, sem)
        pltpu.async_copy(d_hbm.at[pl.ds(bs, K)], di.at[0], sem)

    def wait_loads(buf):
        (si, di, sem) = (buf[0], buf[1], buf[5])
        pltpu.make_async_copy(s_hbm.at[pl.ds(0, K)], si, sem).wait()
        pltpu.make_async_copy(d_hbm.at[pl.ds(0, K)], di.at[0], sem).wait()

    def issue_gathers(buf):
        (si, di, ar, dr, sem) = (buf[0], buf[1], buf[2], buf[3], buf[6])
        pltpu.async_copy(as_hbm.at[si], ar, sem)
        pltpu.async_copy(ad_hbm.at[di.at[0]], dr, sem)

    def wait_gathers(buf):
        (si, di, ar, dr, sem) = (buf[0], buf[1], buf[2], buf[3], buf[6])
        pltpu.make_async_copy(as_hbm.at[si], ar, sem).wait()
        pltpu.make_async_copy(ad_hbm.at[di.at[0]], dr, sem).wait()

    def body(t, b):
        cur = bufs[b]
        (si, di, ar, dr, ec, semL_, semG_, semE_) = cur
        wait_gathers(cur)

        @pl.loop(0, K, unroll=16)
        def _(j):
            a = ar[j] + dr[j]
            t_ = jnp.maximum(a, a * 0.2)
            exr[j] = jnp.exp(t_ - bv)

        pltpu.sync_copy(exr, den_acc.at[di.at[0]], add=True)

        @pl.when(t >= NBUF)
        def _():
            pltpu.make_async_copy(ec, ex_hbm.at[pl.ds(0, 4 * K)], semE_).wait()

        @pl.loop(0, K // 4, unroll=4)
        def _(g):
            v = plsc.load_gather(exr, [4 * g + row_pat, col_pat])
            ec[pl.ds(16 * g, 16)] = v

        bs = wid * (nb * K) + t * K
        pltpu.async_copy(ec, ex_hbm.at[pl.ds(4 * bs, 4 * K)], semE_)

        @pl.when(t + NBUF < nb)
        def _():
            issue_loads(t + NBUF, cur)

        @pl.when(t + NBUF - 1 < nb)
        def _():
            nxt = bufs[(b + NBUF - 1) % NBUF]
            wait_loads(nxt)
            issue_gathers(nxt)

    for b in range(NBUF):
        issue_loads(b, bufs[b])
    for b in range(NBUF - 1):
        wait_loads(bufs[b])
        issue_gathers(bufs[b])

    @pl.loop(0, nb // NBUF)
    def _(tp):
        for b in range(NBUF):
            body(NBUF * tp + b, b)

    for b in range(NBUF):
        pltpu.make_async_copy(exc[b], ex_hbm.at[pl.ds(0, 4 * K)],
                              semE[b]).wait()

    plsc.subcore_barrier()
    off = sid * STRIPE
    pltpu.sync_copy(den_acc.at[pl.ds(off, STRIPE)],
                    den_hbm.at[pl.ds(cid * R + off, STRIPE)])


@jax.jit
def _edge_ex(s, d, as_tab, ad_tab, bound, z16):
    return pl.kernel(
        _edge_ex_kernel,
        out_type=[
            jax.ShapeDtypeStruct((4 * EP,), jnp.float32),
            jax.ShapeDtypeStruct((NCORE * R, 16), jnp.float32),
        ],
        mesh=_MESH,
        scratch_types=[
            [pltpu.VMEM((K,), jnp.int32) for _ in range(NBUF)],
            [pltpu.VMEM((1, K), jnp.int32) for _ in range(NBUF)],
            [pltpu.VMEM((K, 16), jnp.float32) for _ in range(NBUF)],
            [pltpu.VMEM((K, 16), jnp.float32) for _ in range(NBUF)],
            [pltpu.VMEM((4 * K,), jnp.float32) for _ in range(NBUF)],
            pltpu.VMEM((K, 16), jnp.float32),
            pltpu.VMEM((16,), jnp.float32),
            [pltpu.SemaphoreType.DMA for _ in range(NBUF)],
            [pltpu.SemaphoreType.DMA for _ in range(NBUF)],
            [pltpu.SemaphoreType.DMA for _ in range(NBUF)],
            pltpu.VMEM_SHARED((R, 16), jnp.float32),
        ],
        compiler_params=_SC_PARAMS,
    )(s, d, as_tab, ad_tab, bound, z16)


KB = 256
NBUF = 4
_NBLK_B = EP // (NSUB * KB)


def _aggregate_kernel(s_hbm, d_hbm, ex_hbm, x0_hbm, x1_hbm, x2_hbm, x3_hbm,
                      z32_hbm, out_hbm, sidx, didx, msg, exb, semL, semG,
                      out_acc):
    cid = lax.axis_index("c")
    sid = lax.axis_index("s")
    nb = _NBLK_B
    off = sid * STRIPE
    bufs = [(sidx[b], didx[b], msg[b], exb[b], semL[b], semG[b])
            for b in range(NBUF)]

    def issue_loads(t, buf):
        (si, di, eb, sem) = (buf[0], buf[1], buf[3], buf[4])
        bs = sid * (nb * KB) + t * KB
        pltpu.async_copy(s_hbm.at[pl.ds(bs, KB)], si, sem)
        pltpu.async_copy(d_hbm.at[pl.ds(bs, KB)], di.at[0], sem)
        pltpu.async_copy(ex_hbm.at[pl.ds(4 * bs, 4 * KB)], eb, sem)

    def wait_loads(buf):
        (si, di, eb, sem) = (buf[0], buf[1], buf[3], buf[4])
        pltpu.make_async_copy(s_hbm.at[pl.ds(0, KB)], si, sem).wait()
        pltpu.make_async_copy(d_hbm.at[pl.ds(0, KB)], di.at[0], sem).wait()
        pltpu.make_async_copy(ex_hbm.at[pl.ds(0, 4 * KB)], eb, sem).wait()

    def one_pass(hbase, xa_hbm, xb_hbm):
        head = hbase + cid

        def issue_gather(buf):
            (si, mg, sem) = (buf[0], buf[2], buf[5])

            @pl.when(cid == 0)
            def _():
                pltpu.async_copy(xa_hbm.at[si], mg, sem)

            @pl.when(cid == 1)
            def _():
                pltpu.async_copy(xb_hbm.at[si], mg, sem)

        def wait_gather(buf):
            (si, mg, sem) = (buf[0], buf[2], buf[5])
            pltpu.make_async_copy(xa_hbm.at[si], mg, sem).wait()

        def body(t, b):
            cur = bufs[b]
            (si, di, mg, eb, semL_, semG_) = cur
            wait_gather(cur)

            @pl.loop(0, KB, unroll=16)
            def _(j):
                ebc = plsc.load_gather(
                    eb, [jnp.full((16,), 4 * j, jnp.int32) + head])
                mg[j, pl.ds(0, 16)] = mg[j, pl.ds(0, 16)] * ebc
                mg[j, pl.ds(16, 16)] = mg[j, pl.ds(16, 16)] * ebc

            pltpu.sync_copy(mg, out_acc.at[di.at[0]], add=True)

            @pl.when(t + NBUF < nb)
            def _():
                issue_loads(t + NBUF, cur)

            @pl.when(t + NBUF - 1 < nb)
            def _():
                nxt = bufs[(b + NBUF - 1) % NBUF]
                wait_loads(nxt)
                issue_gather(nxt)

        for b in range(NBUF):
            issue_loads(b, bufs[b])
        for b in range(NBUF - 1):
            wait_loads(bufs[b])
            issue_gather(bufs[b])

        @pl.loop(0, nb // NBUF)
        def _(tp):
            for b in range(NBUF):
                body(NBUF * tp + b, b)

        plsc.subcore_barrier()
        pltpu.sync_copy(out_acc.at[pl.ds(off, STRIPE)],
                        out_hbm.at[pl.ds((hbase + cid) * R + off, STRIPE)])
        plsc.subcore_barrier()

    pltpu.sync_copy(z32_hbm, out_acc.at[pl.ds(off, STRIPE)])
    plsc.subcore_barrier()
    one_pass(0, x0_hbm, x1_hbm)
    pltpu.sync_copy(z32_hbm, out_acc.at[pl.ds(off, STRIPE)])
    plsc.subcore_barrier()
    one_pass(2, x2_hbm, x3_hbm)


@jax.jit
def _aggregate(s, d, ex, x0, x1, x2, x3, z32):
    return pl.kernel(
        _aggregate_kernel,
        out_type=jax.ShapeDtypeStruct((HEADS * R, 32), jnp.float32),
        mesh=_MESH,
        scratch_types=[
            [pltpu.VMEM((KB,), jnp.int32) for _ in range(NBUF)],
            [pltpu.VMEM((1, KB), jnp.int32) for _ in range(NBUF)],
            [pltpu.VMEM((KB, 32), jnp.float32) for _ in range(NBUF)],
            [pltpu.VMEM((4 * KB,), jnp.float32) for _ in range(NBUF)],
            [pltpu.SemaphoreType.DMA for _ in range(NBUF)],
            [pltpu.SemaphoreType.DMA for _ in range(NBUF)],
            pltpu.VMEM_SHARED((R, 32), jnp.float32),
        ],
        compiler_params=_SC_PARAMS,
    )(s, d, ex, x0, x1, x2, x3, z32)


def _gat_layer(h, s, d, p, i, z16, z32):
    w, a_s, a_d, b = p[f"W{i}"], p[f"as{i}"], p[f"ad{i}"], p[f"b{i}"]
    x0, x1, x2, x3, as_tab, ad_tab = _dense_stage(h, w, _pack_attn(a_s, a_d))

    z = as_tab[:NT, :HEADS].max(axis=0) + ad_tab[:NT, :HEADS].max(axis=0)
    bound4 = jnp.maximum(z, 0.2 * z)
    bound = jnp.concatenate([bound4, jnp.full((12,), 88.0, jnp.float32)])

    ex, den_parts = _edge_ex(s, d, as_tab, ad_tab, bound, z16)
    den = den_parts[:R][:NT, :HEADS] + den_parts[R:][:NT, :HEADS]

    o = _aggregate(s, d, ex, x0, x1, x2, x3, z32)

    inv = 1.0 / (den + 1e-16)
    heads = [o[h * R : h * R + NT] * inv[:, h : h + 1] for h in range(HEADS)]
    return jnp.concatenate(heads, axis=1) + b


def _gnorm_relu(h, g, bt, ms, res=None):
    hb = h.reshape(B, N, -1)
    mean = hb.mean(axis=1, keepdims=True)
    hc = hb - ms * mean
    var = (hc * hc).mean(axis=1, keepdims=True)
    out = g * hc / jnp.sqrt(var + 1e-5) + bt
    out = out.reshape(NT, -1)
    if res is not None:
        out = out + res
    return jax.nn.relu(out)


def kernel(x, edge_index, params):
    p = params
    xt = jnp.transpose(x, (0, 2, 1)).reshape(NT, D)
    off = jnp.arange(B, dtype=edge_index.dtype) * N
    src = (edge_index[0][None, :] + off[:, None]).reshape(-1)
    dst = (edge_index[1][None, :] + off[:, None]).reshape(-1)
    loop = jnp.arange(NT, dtype=src.dtype)
    pad = jnp.full((EP - ETOT,), SPARE, jnp.int32)
    s = jnp.concatenate([src, loop, pad])
    d = jnp.concatenate([dst, loop, pad])
    z16 = jnp.zeros((STRIPE, 16), jnp.float32)
    z32 = jnp.zeros((STRIPE, 32), jnp.float32)

    h = _gat_layer(xt, s, d, p, 1, z16, z32)
    h = _gnorm_relu(h, p["g1"], p["bt1"], p["ms1"])
    h0 = h
    h = _gat_layer(h0, s, d, p, 2, z16, z32)
    h = _gnorm_relu(h, p["g2"], p["bt2"], p["ms2"], res=h0)
    h0 = h
    h = _gat_layer(h0, s, d, p, 3, z16, z32)
    h = _gnorm_relu(h, p["g3"], p["bt3"], p["ms3"], res=h0)

    pooled = h.reshape(B, N, HID).sum(axis=1) / float(N)
    return pooled @ p["Wc"] + p["bc"]
